# baseline (speedup 1.0000x reference)
import functools

import jax
import jax.numpy as jnp
from jax.experimental import pallas as pl
from jax.experimental.pallas import tpu as pltpu


def _rot_stack_weights(w, b, cin_pad, kw_pad):
    cout, cin, kh, kw = w.shape
    wp = jnp.pad(w, ((0, 0), (0, cin_pad - cin), (0, 0), (0, 0)))
    blocks = []
    for k in range(4):
        wr = jnp.rot90(wp, k=-k, axes=(2, 3))
        wr = jnp.pad(wr, ((0, 0), (0, 0), (0, 0), (0, kw_pad - kw)))
        blocks.append(wr.transpose(2, 3, 1, 0).reshape(kh * kw_pad * cin_pad, cout))
    wm = jnp.concatenate(blocks, axis=1).astype(jnp.bfloat16)
    return wm, jnp.tile(b, 4).reshape(1, 4 * cout)


def _parity_split_weights(w, b):
    cout, cin, kh, kw = w.shape
    wp = jnp.pad(w, ((0, 0), (0, 8 - cin), (0, 0), (0, 0)))
    rots = jnp.stack([jnp.rot90(wp, k=-k, axes=(2, 3)) for k in range(4)])
    w6 = jnp.pad(rots.transpose(3, 4, 2, 0, 1).reshape(kh, kw, 8, 4 * cout),
                 ((0, 0), (0, 1), (0, 0), (0, 0)))
    wd = jnp.zeros((kh, 3, 2, 8, 32), jnp.float32)
    for a in range(kh):
        for s in range(3):
            wd = wd.at[a, s, 0, :, 0:16].set(w6[a, 2 * s])
            wd = wd.at[a, s, 1, :, 0:16].set(w6[a, 2 * s + 1])
            wd = wd.at[a, s, 1, :, 16:32].set(w6[a, 2 * s])
            if s > 0:
                wd = wd.at[a, s, 0, :, 16:32].set(w6[a, 2 * s - 1])
    return wd.reshape(240, 32).astype(jnp.bfloat16), jnp.tile(b, 4).reshape(1, 16)


def _stage1_body(x_ref, w_ref, b_ref, o_ref, xt_ref, pt_ref, *, nimg):
    bvec = b_ref[...].reshape(1, 1, 16)
    for i in range(nimg):
        pb = i % 2
        v = x_ref[i].astype(jnp.bfloat16).reshape(6, 512)
        ev = jnp.pad(v[0:3], ((0, 5), (0, 64)))
        ov = jnp.pad(v[3:6], ((0, 5), (0, 64)))
        xt_ref[pb] = jnp.concatenate([ev, ov], axis=0)
        for t in range(15):
            a, s = divmod(t, 3)
            sh = 16 * a + s
            pt_ref[pb, 16 * t:16 * (t + 1), :] = xt_ref[pb, :, sh:sh + 448]
        acc = jax.lax.dot_general(
            pt_ref[pb], w_ref[...],
            dimension_numbers=(((0,), (0,)), ((), ())),
            preferred_element_type=jnp.float32)
        a6 = acc.reshape(14, 2, 16, 32)
        t4 = jnp.maximum(a6[:, 0], a6[:, 1])
        u = jnp.maximum(t4[:, :, 0:16], t4[:, :, 16:32])
        u = u[:, 0:14]
        o_ref[i] = jnp.maximum(u + bvec, 0.0).reshape(196, 16)


def _stage1(x_flat, w_mat, b_vec, *, batch_blk=32):
    n = x_flat.shape[0]
    return pl.pallas_call(
        functools.partial(_stage1_body, nimg=batch_blk),
        grid=(n // batch_blk,),
        in_specs=[
            pl.BlockSpec((batch_blk, 2, 3, 512), lambda i: (i, 0, 0, 0)),
            pl.BlockSpec((240, 32), lambda i: (0, 0)),
            pl.BlockSpec((1, 16), lambda i: (0, 0)),
        ],
        out_specs=pl.BlockSpec((batch_blk, 196, 16), lambda i: (i, 0, 0)),
        out_shape=jax.ShapeDtypeStruct((n, 196, 16), jnp.float32),
        scratch_shapes=[pltpu.VMEM((2, 16, 576), jnp.bfloat16),
                        pltpu.VMEM((2, 240, 448), jnp.bfloat16)],
        compiler_params=pltpu.CompilerParams(dimension_semantics=("parallel",)),
    )(x_flat, w_mat, b_vec)


def _stage2_body(z_ref, w_ref, b_ref, o_ref, pt_ref, *, nimg):
    bvec = b_ref[...].reshape(1, 1, 64)
    for i in range(nimg):
        pb = i % 2
        zt = z_ref[i]
        for p in range(25):
            a, bb = divmod(p, 5)
            sh = a * 14 + bb
            pt_ref[pb, 16 * p:16 * (p + 1), :] = zt[:, sh:sh + 168]
        acc = jax.lax.dot_general(
            pt_ref[pb], w_ref[...],
            dimension_numbers=(((0,), (0,)), ((), ())),
            preferred_element_type=jnp.float32)
        a5 = acc[0:140].reshape(5, 2, 7, 2, 64)
        u = jnp.maximum(jnp.maximum(a5[:, 0, :, 0], a5[:, 1, :, 0]),
                        jnp.maximum(a5[:, 0, :, 1], a5[:, 1, :, 1]))
        u = u[:, 0:5]
        o_ref[i] = jnp.maximum(u + bvec, 0.0).reshape(25, 64)


def _stage2(z, w_mat, b_vec, *, batch_blk=32):
    n = z.shape[0]
    return pl.pallas_call(
        functools.partial(_stage2_body, nimg=batch_blk),
        grid=(n // batch_blk,),
        in_specs=[
            pl.BlockSpec((batch_blk, 16, 256), lambda i: (i, 0, 0)),
            pl.BlockSpec((400, 64), lambda i: (0, 0)),
            pl.BlockSpec((1, 64), lambda i: (0, 0)),
        ],
        out_specs=pl.BlockSpec((batch_blk, 25, 64), lambda i: (i, 0, 0)),
        out_shape=jax.ShapeDtypeStruct((n, 25, 64), jnp.float32),
        scratch_shapes=[pltpu.VMEM((2, 400, 168), jnp.bfloat16)],
        compiler_params=pltpu.CompilerParams(dimension_semantics=("parallel",)),
    )(z, w_mat, b_vec)


def _mlp_body(x_ref, w1_ref, b1_ref, w2_ref, b2_ref, w3_ref, b3_ref, o_ref):
    h = jnp.dot(x_ref[...], w1_ref[...], preferred_element_type=jnp.float32)
    h = jnp.maximum(h + b1_ref[...], 0.0)
    h = jnp.dot(h, w2_ref[...], preferred_element_type=jnp.float32)
    h = jnp.maximum(h + b2_ref[...], 0.0)
    o_ref[...] = jnp.dot(h, w3_ref[...], preferred_element_type=jnp.float32) + b3_ref[...]


def _mlp(x, w1, b1, w2, b2, w3, b3, *, block_m=512):
    n0, k1 = x.shape
    n = (n0 + block_m - 1) // block_m * block_m
    if n != n0:
        x = jnp.pad(x, ((0, n - n0), (0, 0)))
    d1, d2, d3 = w1.shape[1], w2.shape[1], w3.shape[1]
    out = pl.pallas_call(
        _mlp_body,
        grid=(n // block_m,),
        in_specs=[
            pl.BlockSpec((block_m, k1), lambda i: (i, 0)),
            pl.BlockSpec((k1, d1), lambda i: (0, 0)),
            pl.BlockSpec((1, d1), lambda i: (0, 0)),
            pl.BlockSpec((d1, d2), lambda i: (0, 0)),
            pl.BlockSpec((1, d2), lambda i: (0, 0)),
            pl.BlockSpec((d2, d3), lambda i: (0, 0)),
            pl.BlockSpec((1, d3), lambda i: (0, 0)),
        ],
        out_specs=pl.BlockSpec((block_m, d3), lambda i: (i, 0)),
        out_shape=jax.ShapeDtypeStruct((n, d3), jnp.float32),
        compiler_params=pltpu.CompilerParams(dimension_semantics=("parallel",)),
    )(x, w1, b1, w2, b2, w3, b3)
    return out[:n0]


def kernel(x, conv1_w, conv1_b, conv2_w, conv2_b,
           fc1_w, fc1_b, fc2_w, fc2_b, fc3_w, fc3_b):
    n = x.shape[0]

    w1m, b1v = _parity_split_weights(conv1_w, conv1_b)
    xp = x.reshape(n, 3, 512, 2).transpose(0, 3, 1, 2)
    s1 = _stage1(xp, w1m, b1v)

    z = s1.reshape(n, 14, 14, 16)
    z = jnp.concatenate(
        [jnp.rot90(z[..., 4 * k:4 * (k + 1)], k=k, axes=(1, 2)) for k in range(4)],
        axis=-1)
    z = z.transpose(0, 3, 1, 2).reshape(n, 16, 196)
    z = jnp.pad(z, ((0, 0), (0, 0), (0, 60))).astype(jnp.bfloat16)

    w2m, b2v = _rot_stack_weights(conv2_w, conv2_b, 16, 5)
    s2 = _stage2(z, w2m, b2v)

    xm = s2.transpose(0, 2, 1).reshape(n, 1600)

    w1r = fc1_w.T.reshape(4, 16, 5, 5, 120)
    w1r = jnp.stack([jnp.rot90(w1r[k], k=-k, axes=(1, 2)) for k in range(4)])
    w1 = jnp.pad(w1r.reshape(1600, 120), ((0, 0), (0, 136)))
    b1 = jnp.pad(fc1_b, (0, 136)).reshape(1, 256)
    w2 = jnp.pad(fc2_w.T, ((0, 136), (0, 168)))
    b2 = jnp.pad(fc2_b, (0, 168)).reshape(1, 768)
    w3 = jnp.pad(fc3_w.T, ((0, 168), (0, 156)))
    b3 = jnp.pad(fc3_b, (0, 156)).reshape(1, 256)
    out = _mlp(xm, w1, b1, w2, b2, w3, b3)
    return out[:, :100]


# speedup vs baseline: 1.0093x; 1.0093x over previous
import functools

import jax
import jax.numpy as jnp
from jax.experimental import pallas as pl
from jax.experimental.pallas import tpu as pltpu


def _rot_stack_weights(w, b, cin_pad, kw_pad):
    cout, cin, kh, kw = w.shape
    wp = jnp.pad(w, ((0, 0), (0, cin_pad - cin), (0, 0), (0, 0)))
    blocks = []
    for k in range(4):
        wr = jnp.rot90(wp, k=-k, axes=(2, 3))
        wr = jnp.pad(wr, ((0, 0), (0, 0), (0, 0), (0, kw_pad - kw)))
        blocks.append(wr.transpose(2, 3, 1, 0).reshape(kh * kw_pad * cin_pad, cout))
    wm = jnp.concatenate(blocks, axis=1).astype(jnp.bfloat16)
    return wm, jnp.tile(b, 4).reshape(1, 4 * cout)


def _parity_split_weights(w, b):
    cout, cin, kh, kw = w.shape
    wp = jnp.pad(w, ((0, 0), (0, 8 - cin), (0, 0), (0, 0)))
    rots = jnp.stack([jnp.rot90(wp, k=-k, axes=(2, 3)) for k in range(4)])
    w6 = jnp.pad(rots.transpose(3, 4, 2, 0, 1).reshape(kh, kw, 8, 4 * cout),
                 ((0, 0), (0, 1), (0, 0), (0, 0)))
    wd = jnp.zeros((kh, 3, 2, 8, 32), jnp.float32)
    for a in range(kh):
        for s in range(3):
            wd = wd.at[a, s, 0, :, 0:16].set(w6[a, 2 * s])
            wd = wd.at[a, s, 1, :, 0:16].set(w6[a, 2 * s + 1])
            wd = wd.at[a, s, 1, :, 16:32].set(w6[a, 2 * s])
            if s > 0:
                wd = wd.at[a, s, 0, :, 16:32].set(w6[a, 2 * s - 1])
    return wd.reshape(240, 32).astype(jnp.bfloat16), jnp.tile(b, 4).reshape(1, 16)


def _stage1_body(x_ref, w_ref, b_ref, o_ref, xt_ref, pt_ref, *, nimg):
    bvec = b_ref[...].reshape(1, 1, 16)
    for i in range(nimg):
        pb = i % 2
        v = x_ref[i].astype(jnp.bfloat16).reshape(6, 512)
        ev = jnp.pad(v[0:3], ((0, 5), (0, 64)))
        ov = jnp.pad(v[3:6], ((0, 5), (0, 64)))
        xt_ref[pb] = jnp.concatenate([ev, ov], axis=0)
        for t in range(15):
            a, s = divmod(t, 3)
            sh = 16 * a + s
            pt_ref[pb, 16 * t:16 * (t + 1), :] = xt_ref[pb, :, sh:sh + 448]
        acc = jax.lax.dot_general(
            pt_ref[pb], w_ref[...],
            dimension_numbers=(((0,), (0,)), ((), ())),
            preferred_element_type=jnp.float32)
        a6 = acc.reshape(14, 2, 16, 32)
        t4 = jnp.maximum(a6[:, 0], a6[:, 1])
        u = jnp.maximum(t4[:, :, 0:16], t4[:, :, 16:32])
        u = u[:, 0:14]
        o_ref[i] = jnp.maximum(u + bvec, 0.0).astype(jnp.bfloat16).reshape(196, 16)


def _stage1(x_flat, w_mat, b_vec, *, batch_blk=64):
    n = x_flat.shape[0]
    return pl.pallas_call(
        functools.partial(_stage1_body, nimg=batch_blk),
        grid=(n // batch_blk,),
        in_specs=[
            pl.BlockSpec((batch_blk, 2, 3, 512), lambda i: (i, 0, 0, 0)),
            pl.BlockSpec((240, 32), lambda i: (0, 0)),
            pl.BlockSpec((1, 16), lambda i: (0, 0)),
        ],
        out_specs=pl.BlockSpec((batch_blk, 196, 16), lambda i: (i, 0, 0)),
        out_shape=jax.ShapeDtypeStruct((n, 196, 16), jnp.bfloat16),
        scratch_shapes=[pltpu.VMEM((2, 16, 576), jnp.bfloat16),
                        pltpu.VMEM((2, 240, 448), jnp.bfloat16)],
        compiler_params=pltpu.CompilerParams(dimension_semantics=("parallel",)),
    )(x_flat, w_mat, b_vec)


def _stage2_body(z_ref, w_ref, b_ref, o_ref, pt_ref, *, nimg):
    bvec = b_ref[...].reshape(1, 1, 64)
    for i in range(nimg):
        pb = i % 2
        zt = z_ref[i]
        for p in range(25):
            a, bb = divmod(p, 5)
            sh = a * 14 + bb
            pt_ref[pb, 16 * p:16 * (p + 1), :] = zt[:, sh:sh + 168]
        acc = jax.lax.dot_general(
            pt_ref[pb], w_ref[...],
            dimension_numbers=(((0,), (0,)), ((), ())),
            preferred_element_type=jnp.float32)
        a5 = acc[0:140].reshape(5, 2, 7, 2, 64)
        u = jnp.maximum(jnp.maximum(a5[:, 0, :, 0], a5[:, 1, :, 0]),
                        jnp.maximum(a5[:, 0, :, 1], a5[:, 1, :, 1]))
        u = u[:, 0:5]
        o_ref[i] = jnp.maximum(u + bvec, 0.0).reshape(25, 64)


def _stage2(z, w_mat, b_vec, *, batch_blk=64):
    n = z.shape[0]
    return pl.pallas_call(
        functools.partial(_stage2_body, nimg=batch_blk),
        grid=(n // batch_blk,),
        in_specs=[
            pl.BlockSpec((batch_blk, 16, 256), lambda i: (i, 0, 0)),
            pl.BlockSpec((400, 64), lambda i: (0, 0)),
            pl.BlockSpec((1, 64), lambda i: (0, 0)),
        ],
        out_specs=pl.BlockSpec((batch_blk, 25, 64), lambda i: (i, 0, 0)),
        out_shape=jax.ShapeDtypeStruct((n, 25, 64), jnp.float32),
        scratch_shapes=[pltpu.VMEM((2, 400, 168), jnp.bfloat16)],
        compiler_params=pltpu.CompilerParams(dimension_semantics=("parallel",)),
    )(z, w_mat, b_vec)


def _mlp_body(x_ref, w1_ref, b1_ref, w2_ref, b2_ref, w3_ref, b3_ref, o_ref):
    h = jnp.dot(x_ref[...], w1_ref[...], preferred_element_type=jnp.float32)
    h = jnp.maximum(h + b1_ref[...], 0.0)
    h = jnp.dot(h, w2_ref[...], preferred_element_type=jnp.float32)
    h = jnp.maximum(h + b2_ref[...], 0.0)
    o_ref[...] = jnp.dot(h, w3_ref[...], preferred_element_type=jnp.float32) + b3_ref[...]


def _mlp(x, w1, b1, w2, b2, w3, b3, *, block_m=512):
    n0, k1 = x.shape
    n = (n0 + block_m - 1) // block_m * block_m
    if n != n0:
        x = jnp.pad(x, ((0, n - n0), (0, 0)))
    d1, d2, d3 = w1.shape[1], w2.shape[1], w3.shape[1]
    out = pl.pallas_call(
        _mlp_body,
        grid=(n // block_m,),
        in_specs=[
            pl.BlockSpec((block_m, k1), lambda i: (i, 0)),
            pl.BlockSpec((k1, d1), lambda i: (0, 0)),
            pl.BlockSpec((1, d1), lambda i: (0, 0)),
            pl.BlockSpec((d1, d2), lambda i: (0, 0)),
            pl.BlockSpec((1, d2), lambda i: (0, 0)),
            pl.BlockSpec((d2, d3), lambda i: (0, 0)),
            pl.BlockSpec((1, d3), lambda i: (0, 0)),
        ],
        out_specs=pl.BlockSpec((block_m, d3), lambda i: (i, 0)),
        out_shape=jax.ShapeDtypeStruct((n, d3), jnp.float32),
        compiler_params=pltpu.CompilerParams(dimension_semantics=("parallel",)),
    )(x, w1, b1, w2, b2, w3, b3)
    return out[:n0]


def kernel(x, conv1_w, conv1_b, conv2_w, conv2_b,
           fc1_w, fc1_b, fc2_w, fc2_b, fc3_w, fc3_b):
    n = x.shape[0]

    w1m, b1v = _parity_split_weights(conv1_w, conv1_b)
    xp = x.reshape(n, 3, 512, 2).transpose(0, 3, 1, 2)
    s1 = _stage1(xp, w1m, b1v)

    z = s1.reshape(n, 14, 14, 16)
    z = jnp.concatenate(
        [jnp.rot90(z[..., 4 * k:4 * (k + 1)], k=k, axes=(1, 2)) for k in range(4)],
        axis=-1)
    z = z.transpose(0, 3, 1, 2).reshape(n, 16, 196)
    z = jnp.pad(z, ((0, 0), (0, 0), (0, 60))).astype(jnp.bfloat16)

    w2m, b2v = _rot_stack_weights(conv2_w, conv2_b, 16, 5)
    s2 = _stage2(z, w2m, b2v)

    xm = s2.transpose(0, 2, 1).reshape(n, 1600)

    w1r = fc1_w.T.reshape(4, 16, 5, 5, 120)
    w1r = jnp.stack([jnp.rot90(w1r[k], k=-k, axes=(1, 2)) for k in range(4)])
    w1 = jnp.pad(w1r.reshape(1600, 120), ((0, 0), (0, 136)))
    b1 = jnp.pad(fc1_b, (0, 136)).reshape(1, 256)
    w2 = jnp.pad(fc2_w.T, ((0, 136), (0, 168)))
    b2 = jnp.pad(fc2_b, (0, 168)).reshape(1, 768)
    w3 = jnp.pad(fc3_w.T, ((0, 168), (0, 156)))
    b3 = jnp.pad(fc3_b, (0, 156)).reshape(1, 256)
    out = _mlp(xm, w1, b1, w2, b2, w3, b3)
    return out[:, :100]
